# baseline (device time: 655117 ns/iter reference)
import jax
import jax.numpy as jnp
from jax import lax
from jax.experimental import pallas as pl
from jax.experimental.pallas import tpu as pltpu

N_DEV = 4
BM = 1024
NC = 2
MESH = pl.DeviceIdType.MESH


def kernel(x, w_mat, scale_x, scale_w):
    m, k = x.shape
    _, n = w_mat.shape
    hl = n // 2
    sc = hl // NC
    scale = (scale_x * scale_w).astype(jnp.float32)

    def body(scale_ref, x_ref, wm_ref, out_ref, rs_recv, w_ref, w0_ref,
             agd, va, vb16, vc16, rs_send_sems, rs_recv_sems,
             ag_send_sems, ag_recv_sems, cp_sems):
        my = lax.axis_index("i")
        left = lax.rem(my + (N_DEV - 1), N_DEV)
        right = lax.rem(my + 1, N_DEV)

        barrier = pltpu.get_barrier_semaphore()
        for nbr in (left, right):
            pl.semaphore_signal(barrier, inc=1, device_id=(nbr,),
                                device_id_type=MESH)
        pl.semaphore_wait(barrier, 2)

        def rows(b):
            return pl.ds(b * BM, BM)

        def cols(d, c):
            return pl.ds(d * hl + c * sc, sc)

        def mm(b, d, c):
            a16 = x_ref[rows(b), :].astype(jnp.bfloat16)
            b16 = wm_ref[:, cols(d, c)].astype(jnp.bfloat16)
            return lax.dot_general(
                a16, b16, (((1,), (0,)), ((), ())),
                preferred_element_type=jnp.float32,
            )

        def rs_start(s, d, c):
            tgt = right if d == 0 else left
            src = (w0_ref if s == 0 else w_ref).at[:, cols(d, c)]
            k_ = (s * 2 + d) * NC + c
            r = pltpu.make_async_remote_copy(
                src_ref=src,
                dst_ref=rs_recv.at[s, :, cols(d, c)],
                send_sem=rs_send_sems.at[k_],
                recv_sem=rs_recv_sems.at[k_],
                device_id=(tgt,),
                device_id_type=MESH,
            )
            r.start()
            return r

        rdmas = {}
        for c in range(NC):
          for d in (0, 1):
            va[...] = mm(my, d, c)
            vc16[...] = va[...].astype(jnp.bfloat16)
            cpo = pltpu.make_async_copy(vc16, w0_ref.at[:, cols(d, c)],
                                        cp_sems.at[2])
            cpo.start()
            cpo.wait()
            rdmas[(0, d, c)] = rs_start(0, d, c)

        def ag_start(s, d, c):
            tgt = right if d == 0 else left
            if d == 0:
                sb = lax.rem(my + (N_DEV + 1 - s), N_DEV)
            else:
                sb = lax.rem(my + (N_DEV - 1 + s), N_DEV)
            del sb
            k_ = (s * 2 + d) * NC + c
            r = pltpu.make_async_remote_copy(
                src_ref=agd.at[s, :, cols(d, c)],
                dst_ref=agd.at[s + 1, :, cols(d, c)],
                send_sem=ag_send_sems.at[k_],
                recv_sem=ag_recv_sems.at[k_],
                device_id=(tgt,),
                device_id_type=MESH,
            )
            r.start()
            return r

        ag = {}

        for s in range(3):
            bA = lax.rem(my + (2 * N_DEV - 1 - s), N_DEV)
            bB = lax.rem(my + (1 + s), N_DEV)
            for c in range(NC):
              for d in (0, 1):
                rdmas[(s, d, c)].wait()
                b = bA if d == 0 else bB
                cpb = pltpu.make_async_copy(rs_recv.at[s, :, cols(d, c)],
                                            vb16, cp_sems.at[1])
                cpb.start()
                va[...] = mm(b, d, c)
                cpb.wait()
                if s < 2:
                    vc16[...] = (va[...] + vb16[...].astype(jnp.float32)
                                 ).astype(jnp.bfloat16)
                    cpo = pltpu.make_async_copy(
                        vc16, w_ref.at[:, cols(d, c)], cp_sems.at[2])
                    cpo.start()
                    cpo.wait()
                    rdmas[(s + 1, d, c)] = rs_start(s + 1, d, c)
                else:
                    va[...] = (va[...] + vb16[...].astype(jnp.float32)
                               ) * scale_ref[0]
                    vc16[...] = va[...].astype(jnp.bfloat16)
                    cpo = pltpu.make_async_copy(
                        va, out_ref.at[rows(b), cols(d, c)], cp_sems.at[2])
                    cpa = pltpu.make_async_copy(
                        vc16, agd.at[0, :, cols(d, c)], cp_sems.at[0])
                    cpo.start()
                    cpa.start()
                    cpo.wait()
                    cpa.wait()
                    ag[(0, d, c)] = ag_start(0, d, c)

        for s in range(3):
            gA = lax.rem(my + (N_DEV - s), N_DEV)
            gB = lax.rem(my + s, N_DEV)
            for c in range(NC):
              for d in (0, 1):
                ag[(s, d, c)].wait()
                if s < 2:
                    ag[(s + 1, d, c)] = ag_start(s + 1, d, c)
                g = gA if d == 0 else gB
                cpb = pltpu.make_async_copy(agd.at[s + 1, :, cols(d, c)],
                                            vb16, cp_sems.at[1])
                cpb.start()
                cpb.wait()
                va[...] = vb16[...].astype(jnp.float32)
                cpo = pltpu.make_async_copy(
                    va, out_ref.at[rows(g), cols(d, c)], cp_sems.at[2])
                cpo.start()
                cpo.wait()

    out, _, _, _, _ = pl.pallas_call(
        body,
        out_shape=[
            jax.ShapeDtypeStruct((m, n), jnp.float32),
            jax.ShapeDtypeStruct((3, BM, n), jnp.bfloat16),
            jax.ShapeDtypeStruct((BM, n), jnp.bfloat16),
            jax.ShapeDtypeStruct((BM, n), jnp.bfloat16),
            jax.ShapeDtypeStruct((4, BM, n), jnp.bfloat16),
        ],
        in_specs=[
            pl.BlockSpec(memory_space=pltpu.MemorySpace.SMEM),
            pl.BlockSpec(memory_space=pltpu.MemorySpace.VMEM),
            pl.BlockSpec(memory_space=pltpu.MemorySpace.VMEM),
        ],
        out_specs=[
            pl.BlockSpec(memory_space=pl.ANY),
            pl.BlockSpec(memory_space=pl.ANY),
            pl.BlockSpec(memory_space=pl.ANY),
            pl.BlockSpec(memory_space=pl.ANY),
            pl.BlockSpec(memory_space=pl.ANY),
        ],
        scratch_shapes=[
            pltpu.VMEM((BM, sc), jnp.float32),
            pltpu.VMEM((BM, sc), jnp.bfloat16),
            pltpu.VMEM((BM, sc), jnp.bfloat16),
            pltpu.SemaphoreType.DMA((6 * NC,)),
            pltpu.SemaphoreType.DMA((6 * NC,)),
            pltpu.SemaphoreType.DMA((6 * NC,)),
            pltpu.SemaphoreType.DMA((6 * NC,)),
            pltpu.SemaphoreType.DMA((3,)),
        ],
        compiler_params=pltpu.CompilerParams(collective_id=0),
    )(scale, x, w_mat)
    return out


# device time: 647708 ns/iter; 1.0114x vs baseline; 1.0114x over previous
import jax
import jax.numpy as jnp
from jax import lax
from jax.experimental import pallas as pl
from jax.experimental.pallas import tpu as pltpu

N_DEV = 4
BM = 1024
NC = 4
MESH = pl.DeviceIdType.MESH


def kernel(x, w_mat, scale_x, scale_w):
    m, k = x.shape
    _, n = w_mat.shape
    hl = n // 2
    sc = hl // NC
    scale = (scale_x * scale_w).astype(jnp.float32)

    def body(scale_ref, x_ref, wm_ref, out_ref, rs_recv, w_ref, w0_ref,
             agd, va, vb16, vc16, rs_send_sems, rs_recv_sems,
             ag_send_sems, ag_recv_sems, cp_sems):
        my = lax.axis_index("i")
        left = lax.rem(my + (N_DEV - 1), N_DEV)
        right = lax.rem(my + 1, N_DEV)

        barrier = pltpu.get_barrier_semaphore()
        for nbr in (left, right):
            pl.semaphore_signal(barrier, inc=1, device_id=(nbr,),
                                device_id_type=MESH)
        pl.semaphore_wait(barrier, 2)

        def rows(b):
            return pl.ds(b * BM, BM)

        def cols(d, c):
            return pl.ds(d * hl + c * sc, sc)

        def mm(b, d, c):
            a16 = x_ref[rows(b), :].astype(jnp.bfloat16)
            b16 = wm_ref[:, cols(d, c)].astype(jnp.bfloat16)
            return lax.dot_general(
                a16, b16, (((1,), (0,)), ((), ())),
                preferred_element_type=jnp.float32,
            )

        def rs_start(s, d, c):
            tgt = right if d == 0 else left
            src = (w0_ref if s == 0 else w_ref).at[:, cols(d, c)]
            k_ = (s * 2 + d) * NC + c
            r = pltpu.make_async_remote_copy(
                src_ref=src,
                dst_ref=rs_recv.at[s, :, cols(d, c)],
                send_sem=rs_send_sems.at[k_],
                recv_sem=rs_recv_sems.at[k_],
                device_id=(tgt,),
                device_id_type=MESH,
            )
            r.start()
            return r

        rdmas = {}
        for c in range(NC):
          for d in (0, 1):
            va[...] = mm(my, d, c)
            vc16[...] = va[...].astype(jnp.bfloat16)
            cpo = pltpu.make_async_copy(vc16, w0_ref.at[:, cols(d, c)],
                                        cp_sems.at[2])
            cpo.start()
            cpo.wait()
            rdmas[(0, d, c)] = rs_start(0, d, c)

        def ag_start(s, d, c):
            tgt = right if d == 0 else left
            if d == 0:
                sb = lax.rem(my + (N_DEV + 1 - s), N_DEV)
            else:
                sb = lax.rem(my + (N_DEV - 1 + s), N_DEV)
            del sb
            k_ = (s * 2 + d) * NC + c
            r = pltpu.make_async_remote_copy(
                src_ref=agd.at[s, :, cols(d, c)],
                dst_ref=agd.at[s + 1, :, cols(d, c)],
                send_sem=ag_send_sems.at[k_],
                recv_sem=ag_recv_sems.at[k_],
                device_id=(tgt,),
                device_id_type=MESH,
            )
            r.start()
            return r

        ag = {}

        for s in range(3):
            bA = lax.rem(my + (2 * N_DEV - 1 - s), N_DEV)
            bB = lax.rem(my + (1 + s), N_DEV)
            for c in range(NC):
              for d in (0, 1):
                rdmas[(s, d, c)].wait()
                b = bA if d == 0 else bB
                cpb = pltpu.make_async_copy(rs_recv.at[s, :, cols(d, c)],
                                            vb16, cp_sems.at[1])
                cpb.start()
                va[...] = mm(b, d, c)
                cpb.wait()
                if s < 2:
                    vc16[...] = (va[...] + vb16[...].astype(jnp.float32)
                                 ).astype(jnp.bfloat16)
                    cpo = pltpu.make_async_copy(
                        vc16, w_ref.at[:, cols(d, c)], cp_sems.at[2])
                    cpo.start()
                    cpo.wait()
                    rdmas[(s + 1, d, c)] = rs_start(s + 1, d, c)
                else:
                    va[...] = (va[...] + vb16[...].astype(jnp.float32)
                               ) * scale_ref[0]
                    vc16[...] = va[...].astype(jnp.bfloat16)
                    cpo = pltpu.make_async_copy(
                        va, out_ref.at[rows(b), cols(d, c)], cp_sems.at[2])
                    cpa = pltpu.make_async_copy(
                        vc16, agd.at[0, :, cols(d, c)], cp_sems.at[0])
                    cpo.start()
                    cpa.start()
                    cpo.wait()
                    cpa.wait()
                    ag[(0, d, c)] = ag_start(0, d, c)

        for s in range(3):
            gA = lax.rem(my + (N_DEV - s), N_DEV)
            gB = lax.rem(my + s, N_DEV)
            for c in range(NC):
              for d in (0, 1):
                ag[(s, d, c)].wait()
                if s < 2:
                    ag[(s + 1, d, c)] = ag_start(s + 1, d, c)
                g = gA if d == 0 else gB
                cpb = pltpu.make_async_copy(agd.at[s + 1, :, cols(d, c)],
                                            vb16, cp_sems.at[1])
                cpb.start()
                cpb.wait()
                va[...] = vb16[...].astype(jnp.float32)
                cpo = pltpu.make_async_copy(
                    va, out_ref.at[rows(g), cols(d, c)], cp_sems.at[2])
                cpo.start()
                cpo.wait()

    out, _, _, _, _ = pl.pallas_call(
        body,
        out_shape=[
            jax.ShapeDtypeStruct((m, n), jnp.float32),
            jax.ShapeDtypeStruct((3, BM, n), jnp.bfloat16),
            jax.ShapeDtypeStruct((BM, n), jnp.bfloat16),
            jax.ShapeDtypeStruct((BM, n), jnp.bfloat16),
            jax.ShapeDtypeStruct((4, BM, n), jnp.bfloat16),
        ],
        in_specs=[
            pl.BlockSpec(memory_space=pltpu.MemorySpace.SMEM),
            pl.BlockSpec(memory_space=pltpu.MemorySpace.VMEM),
            pl.BlockSpec(memory_space=pltpu.MemorySpace.VMEM),
        ],
        out_specs=[
            pl.BlockSpec(memory_space=pl.ANY),
            pl.BlockSpec(memory_space=pl.ANY),
            pl.BlockSpec(memory_space=pl.ANY),
            pl.BlockSpec(memory_space=pl.ANY),
            pl.BlockSpec(memory_space=pl.ANY),
        ],
        scratch_shapes=[
            pltpu.VMEM((BM, sc), jnp.float32),
            pltpu.VMEM((BM, sc), jnp.bfloat16),
            pltpu.VMEM((BM, sc), jnp.bfloat16),
            pltpu.SemaphoreType.DMA((6 * NC,)),
            pltpu.SemaphoreType.DMA((6 * NC,)),
            pltpu.SemaphoreType.DMA((6 * NC,)),
            pltpu.SemaphoreType.DMA((6 * NC,)),
            pltpu.SemaphoreType.DMA((3,)),
        ],
        compiler_params=pltpu.CompilerParams(collective_id=0),
    )(scale, x, w_mat)
    return out
